# baseline (speedup 1.0000x reference)
import functools

import jax
import jax.numpy as jnp
from jax import lax
from jax.experimental import pallas as pl
from jax.experimental.pallas import tpu as pltpu
from jax.experimental.pallas import tpu_sc as plsc

B = 1024
S = 200
T = 8
D = 32
N = B * S * T

NC = 2
NS = 16
NW = NC * NS

NU = S * T
U_PER_W = NU // NW
HALF = B // 2

R_PAD = 5376
STAGE_ROWS = R_PAD // NS


def _sc_gather(tbl, idx):
    mesh = plsc.VectorSubcoreMesh(core_axis_name="c", subcore_axis_name="s")

    STEPS = 2 * U_PER_W

    @functools.partial(
        pl.kernel,
        out_type=jax.ShapeDtypeStruct((NU, 4, 2, 4096), jnp.float32),
        mesh=mesh,
        scratch_types=[
            pltpu.VMEM_SHARED((R_PAD, D), jnp.float32),
            pltpu.VMEM((4, 4, 128), jnp.int32),
            pltpu.VMEM((4, HALF, D), jnp.float32),
            pltpu.VMEM((2, 4, 4096), jnp.float32),
            pltpu.SemaphoreType.DMA((4,)),
            pltpu.SemaphoreType.DMA((4,)),
            pltpu.SemaphoreType.DMA((2,)),
        ],
        compiler_params=pltpu.CompilerParams(
            use_tc_tiling_on_sc=False, needs_layout_passes=False
        ),
    )
    def k(tbl_hbm, idx_hbm, out_hbm, tbl_sp, idx_v, rows_v, outbuf,
          isem, gsem, osem):
        cid = lax.axis_index("c")
        sid = lax.axis_index("s")
        wid = sid * NC + cid

        stage0 = pl.multiple_of(sid * STAGE_ROWS, 8)
        pltpu.sync_copy(
            tbl_hbm.at[pl.ds(stage0, STAGE_ROWS)],
            tbl_sp.at[pl.ds(stage0, STAGE_ROWS)],
        )
        plsc.subcore_barrier()

        u0 = wid * U_PER_W
        kg0 = wid * STEPS
        iota16 = lax.iota(jnp.int32, 16)

        def issue_idx(kg, s4):
            pltpu.async_copy(idx_hbm.at[kg], idx_v.at[s4], isem.at[s4])

        def wait_idx(s4):
            pltpu.make_async_copy(
                idx_hbm.at[0], idx_v.at[s4], isem.at[s4]
            ).wait()

        def issue_gathers(s4):
            for j in range(4):
                pltpu.async_copy(
                    tbl_sp.at[idx_v.at[s4, j]],
                    rows_v.at[s4, pl.ds(j * 128, 128)],
                    gsem.at[s4],
                )

        def wait_gathers(s4):
            for j in range(4):
                pltpu.make_async_copy(
                    tbl_sp.at[idx_v.at[s4, j]],
                    rows_v.at[s4, pl.ds(j * 128, 128)],
                    gsem.at[s4],
                ).wait()

        def transpose(s4, slot):
            @plsc.parallel_loop(0, 512, 1, unroll=8)
            def tr(i):
                kk = i >> 5
                j = i & 31
                dmod = (iota16 + kk) & 15
                cs2 = (dmod & 7) * 128 + ((j >> 3) * 1024 + (j & 7) * 16
                                          + iota16)
                cd8 = dmod >> 3
                row_idx = iota16 + j * 16
                for dhi in range(2):
                    v = plsc.load_gather(
                        rows_v.at[s4], [row_idx, dmod + dhi * 16]
                    )
                    plsc.store_scatter(
                        outbuf.at[slot], [cd8 + dhi * 2, cs2], v
                    )

        issue_idx(kg0, 0)
        issue_idx(kg0 + 1, 1)
        wait_idx(0)
        issue_gathers(0)

        def outer(g, carry):
            for s in range(4):
                kg = kg0 + g * 4 + s
                u = u0 + g * 2 + (s >> 1)
                half = s & 1
                slot = s % 2

                if s < 2:
                    issue_idx(kg + 2, (s + 2) % 4)
                else:
                    @pl.when(g < U_PER_W // 2 - 1)
                    def _():
                        issue_idx(kg + 2, (s + 2) % 4)

                def _gathers_next():
                    wait_idx((s + 1) % 4)
                    issue_gathers((s + 1) % 4)

                if s < 3:
                    _gathers_next()
                else:
                    @pl.when(g < U_PER_W // 2 - 1)
                    def _():
                        _gathers_next()

                wait_gathers(s)
                if s < 2:
                    @pl.when(g > 0)
                    def _():
                        pltpu.make_async_copy(
                            outbuf.at[slot], out_hbm.at[0, :, 0],
                            osem.at[slot],
                        ).wait()
                else:
                    pltpu.make_async_copy(
                        outbuf.at[slot], out_hbm.at[0, :, 0], osem.at[slot]
                    ).wait()

                transpose(s, slot)
                pltpu.async_copy(
                    outbuf.at[slot], out_hbm.at[u, :, half], osem.at[slot]
                )
            return carry

        lax.fori_loop(0, U_PER_W // 2, outer, 0)
        for slot in range(2):
            pltpu.make_async_copy(
                outbuf.at[slot], out_hbm.at[0, :, 0], osem.at[slot]
            ).wait()

    return k(tbl, idx)


def kernel(x, emb_table_0, emb_table_1, emb_table_2, emb_table_3,
           emb_table_4, emb_table_5, emb_table_6, emb_table_7):
    stacked = jnp.concatenate(
        [
            emb_table_0[:1000],
            emb_table_1[:1000],
            emb_table_2[:1000],
            emb_table_3[:1000],
            emb_table_4[:1000],
            emb_table_5,
            emb_table_6,
            emb_table_7,
            jnp.zeros((R_PAD - 5253, D), jnp.float32),
        ],
        axis=0,
    )

    pos_seq = jnp.arange(S, dtype=jnp.int32) + 5000
    pos_fut = jnp.concatenate(
        [jnp.zeros(S - 50, jnp.int32), jnp.arange(1, 51, dtype=jnp.int32)]
    ) + 5200
    is_fut = jnp.concatenate(
        [jnp.zeros(S - 50, jnp.int32), jnp.ones(50, jnp.int32)]
    ) + 5251
    pos = jnp.stack([pos_seq, pos_fut, is_fut], axis=1)
    bases = jnp.arange(5, dtype=jnp.int32) * 1000
    xt = jnp.transpose(x.astype(jnp.int32), (1, 2, 0)) + bases[None, :, None]
    idx_t = jnp.concatenate(
        [xt, jnp.broadcast_to(pos[:, :, None], (S, 3, B))], axis=1
    )
    idx_arg = idx_t.reshape(2 * NU, 4, 128)

    y = _sc_gather(stacked, idx_arg)
    y2 = y.reshape(S, T, 4, 2, 4, 8, 128)
    out = y2.transpose(3, 4, 6, 0, 1, 2, 5).reshape(B, S, T, D)
    return out


# speedup vs baseline: 1.0483x; 1.0483x over previous
import functools

import jax
import jax.numpy as jnp
from jax import lax
from jax.experimental import pallas as pl
from jax.experimental.pallas import tpu as pltpu
from jax.experimental.pallas import tpu_sc as plsc

B = 1024
S = 200
T = 8
D = 32
N = B * S * T

NC = 2
NS = 16
NW = NC * NS

NU = S * T
U_PER_W = NU // NW
HALF = B // 2

R_PAD = 5376
STAGE_ROWS = R_PAD // NS


def _sc_gather(tbl, idx):
    mesh = plsc.VectorSubcoreMesh(core_axis_name="c", subcore_axis_name="s")

    STEPS = 2 * U_PER_W

    @functools.partial(
        pl.kernel,
        out_type=jax.ShapeDtypeStruct((NU, 4, 2, 4096), jnp.float32),
        mesh=mesh,
        scratch_types=[
            pltpu.VMEM_SHARED((R_PAD, D), jnp.float32),
            pltpu.VMEM((4, 4, 128), jnp.int32),
            pltpu.VMEM((4, HALF, D), jnp.float32),
            pltpu.VMEM((2, 4, 4096), jnp.float32),
            pltpu.SemaphoreType.DMA((4,)),
            pltpu.SemaphoreType.DMA((4,)),
            pltpu.SemaphoreType.DMA((2,)),
        ],
        compiler_params=pltpu.CompilerParams(
            use_tc_tiling_on_sc=False, needs_layout_passes=False
        ),
    )
    def k(tbl_hbm, idx_hbm, out_hbm, tbl_sp, idx_v, rows_v, outbuf,
          isem, gsem, osem):
        cid = lax.axis_index("c")
        sid = lax.axis_index("s")
        wid = sid * NC + cid

        stage0 = pl.multiple_of(sid * STAGE_ROWS, 8)
        pltpu.sync_copy(
            tbl_hbm.at[pl.ds(stage0, STAGE_ROWS)],
            tbl_sp.at[pl.ds(stage0, STAGE_ROWS)],
        )
        plsc.subcore_barrier()

        u0 = wid * U_PER_W
        kg0 = wid * STEPS
        iota16 = lax.iota(jnp.int32, 16)

        def issue_idx(kg, s4):
            pltpu.async_copy(idx_hbm.at[kg], idx_v.at[s4], isem.at[s4])

        def wait_idx(s4):
            pltpu.make_async_copy(
                idx_hbm.at[0], idx_v.at[s4], isem.at[s4]
            ).wait()

        def issue_gathers(s4):
            for j in range(4):
                pltpu.async_copy(
                    tbl_sp.at[idx_v.at[s4, j]],
                    rows_v.at[s4, pl.ds(j * 128, 128)],
                    gsem.at[s4],
                )

        def wait_gathers(s4):
            for j in range(4):
                pltpu.make_async_copy(
                    tbl_sp.at[idx_v.at[s4, j]],
                    rows_v.at[s4, pl.ds(j * 128, 128)],
                    gsem.at[s4],
                ).wait()

        def transpose(s4, slot):
            @plsc.parallel_loop(0, 512, 1, unroll=4)
            def tr(i):
                kk = i >> 5
                j = i & 31
                dmod = (iota16 + kk) & 15
                cs2 = (dmod & 7) * 128 + ((j >> 3) * 1024 + (j & 7) * 16
                                          + iota16)
                cd8 = dmod >> 3
                row_idx = iota16 + j * 16
                for dhi in range(2):
                    v = plsc.load_gather(
                        rows_v.at[s4], [row_idx, dmod + dhi * 16]
                    )
                    plsc.store_scatter(
                        outbuf.at[slot], [cd8 + dhi * 2, cs2], v
                    )

        issue_idx(kg0, 0)
        issue_idx(kg0 + 1, 1)
        wait_idx(0)
        issue_gathers(0)

        def outer(g, carry):
            for s in range(4):
                kg = kg0 + g * 4 + s
                u = u0 + g * 2 + (s >> 1)
                half = s & 1
                slot = s % 2

                if s < 2:
                    issue_idx(kg + 2, (s + 2) % 4)
                else:
                    @pl.when(g < U_PER_W // 2 - 1)
                    def _():
                        issue_idx(kg + 2, (s + 2) % 4)

                def _gathers_next():
                    wait_idx((s + 1) % 4)
                    issue_gathers((s + 1) % 4)

                if s < 3:
                    _gathers_next()
                else:
                    @pl.when(g < U_PER_W // 2 - 1)
                    def _():
                        _gathers_next()

                wait_gathers(s)
                if s < 2:
                    @pl.when(g > 0)
                    def _():
                        pltpu.make_async_copy(
                            outbuf.at[slot], out_hbm.at[0, :, 0],
                            osem.at[slot],
                        ).wait()
                else:
                    pltpu.make_async_copy(
                        outbuf.at[slot], out_hbm.at[0, :, 0], osem.at[slot]
                    ).wait()

                transpose(s, slot)
                pltpu.async_copy(
                    outbuf.at[slot], out_hbm.at[u, :, half], osem.at[slot]
                )
            return carry

        lax.fori_loop(0, U_PER_W // 2, outer, 0)
        for slot in range(2):
            pltpu.make_async_copy(
                outbuf.at[slot], out_hbm.at[0, :, 0], osem.at[slot]
            ).wait()

    return k(tbl, idx)


def kernel(x, emb_table_0, emb_table_1, emb_table_2, emb_table_3,
           emb_table_4, emb_table_5, emb_table_6, emb_table_7):
    stacked = jnp.concatenate(
        [
            emb_table_0[:1000],
            emb_table_1[:1000],
            emb_table_2[:1000],
            emb_table_3[:1000],
            emb_table_4[:1000],
            emb_table_5,
            emb_table_6,
            emb_table_7,
            jnp.zeros((R_PAD - 5253, D), jnp.float32),
        ],
        axis=0,
    )

    pos_seq = jnp.arange(S, dtype=jnp.int32) + 5000
    pos_fut = jnp.concatenate(
        [jnp.zeros(S - 50, jnp.int32), jnp.arange(1, 51, dtype=jnp.int32)]
    ) + 5200
    is_fut = jnp.concatenate(
        [jnp.zeros(S - 50, jnp.int32), jnp.ones(50, jnp.int32)]
    ) + 5251
    pos = jnp.stack([pos_seq, pos_fut, is_fut], axis=1)
    bases = jnp.arange(5, dtype=jnp.int32) * 1000
    xt = jnp.transpose(x.astype(jnp.int32), (1, 2, 0)) + bases[None, :, None]
    idx_t = jnp.concatenate(
        [xt, jnp.broadcast_to(pos[:, :, None], (S, 3, B))], axis=1
    )
    idx_arg = idx_t.reshape(2 * NU, 4, 128)

    y = _sc_gather(stacked, idx_arg)
    y2 = y.reshape(S, T, 4, 2, 4, 8, 128)
    out = y2.transpose(3, 4, 6, 0, 1, 2, 5).reshape(B, S, T, D)
    return out
